# baseline (device time: 23092 ns/iter reference)
import jax
import jax.numpy as jnp
from jax import lax
from jax.experimental import pallas as pl
from jax.experimental.pallas import tpu as pltpu

N_DEV = 16
N_STEPS = 4
EXPERTS_PER_DEV = 2


def kernel(x, router_W, route_idx, expert_W, shared_W):
    n_tok, d_model = x.shape
    n_experts = router_W.shape[1]
    d_ff = expert_W.shape[2]

    def body(x_hbm, router_W_hbm, route_idx_hbm, expert_W_hbm, shared_W_hbm,
             out_ref, x_vm, router_W_vm, route_idx_vm, expert_W_vm,
             shared_W_vm, acc_ref, send_ref, comm_ref,
             copy_sems, send_sems, recv_sems):
        my = lax.axis_index("i")

        barrier_sem = pltpu.get_barrier_semaphore()
        for k in range(N_STEPS):
            partner = my ^ (1 << k)
            pl.semaphore_signal(
                barrier_sem, inc=1,
                device_id=(partner,), device_id_type=pl.DeviceIdType.MESH,
            )

        stage = [
            pltpu.make_async_copy(x_hbm, x_vm, copy_sems.at[0]),
            pltpu.make_async_copy(router_W_hbm, router_W_vm, copy_sems.at[1]),
            pltpu.make_async_copy(route_idx_hbm, route_idx_vm, copy_sems.at[2]),
            pltpu.make_async_copy(expert_W_hbm, expert_W_vm, copy_sems.at[3]),
            pltpu.make_async_copy(shared_W_hbm, shared_W_vm, copy_sems.at[4]),
        ]
        for c in stage:
            c.start()
        for c in stage[:3]:
            c.wait()

        xv = x_vm[...]
        xb = xv.astype(jnp.bfloat16)
        scores = jnp.dot(xb, router_W_vm[...].astype(jnp.bfloat16),
                         preferred_element_type=jnp.float32)
        s_max = jnp.max(scores, axis=-1, keepdims=True)
        p = jnp.exp(scores - s_max)
        probs = p / jnp.sum(p, axis=-1, keepdims=True)

        idx = route_idx_vm[...]
        eids = lax.broadcasted_iota(jnp.int32, (n_tok, n_experts), 1)
        p_tok = jnp.sum(jnp.where(idx == eids, probs, 0.0),
                        axis=-1, keepdims=True)

        stage[3].wait()
        partial = jnp.zeros((n_tok, d_ff), jnp.float32)
        for j in range(EXPERTS_PER_DEV):
            e_glob = my * EXPERTS_PER_DEV + j
            y = jnp.dot(xb, expert_W_vm[j].astype(jnp.bfloat16),
                        preferred_element_type=jnp.float32)
            coef = jnp.where(idx == e_glob, p_tok, 0.0)
            partial = partial + coef * y
        acc_ref[...] = partial
        send_ref[0] = partial.astype(jnp.bfloat16)

        pl.semaphore_wait(barrier_sem, N_STEPS)

        rdmas = []
        for k in range(N_STEPS):
            partner = my ^ (1 << k)
            rdma = pltpu.make_async_remote_copy(
                src_ref=send_ref.at[k],
                dst_ref=comm_ref.at[k],
                send_sem=send_sems.at[k],
                recv_sem=recv_sems.at[k],
                device_id=(partner,),
                device_id_type=pl.DeviceIdType.MESH,
            )
            rdma.start()
            rdmas.append(rdma)
            if k == 0:
                stage[4].wait()
                shared = jnp.dot(xb, shared_W_vm[...].astype(jnp.bfloat16),
                                 preferred_element_type=jnp.float32)
                out_ref[...] = shared
            rdma.wait_recv()
            new_acc = acc_ref[...] + comm_ref[k].astype(jnp.float32)
            acc_ref[...] = new_acc
            if k + 1 < N_STEPS:
                send_ref[k + 1] = new_acc.astype(jnp.bfloat16)

        out_ref[...] = out_ref[...] + acc_ref[...]
        for rdma in rdmas:
            rdma.wait_send()

    return pl.pallas_call(
        body,
        out_shape=jax.ShapeDtypeStruct((n_tok, d_ff), jnp.float32),
        in_specs=[pl.BlockSpec(memory_space=pl.ANY)] * 5,
        out_specs=pl.BlockSpec(memory_space=pltpu.VMEM),
        scratch_shapes=[
            pltpu.VMEM((n_tok, d_model), jnp.float32),
            pltpu.VMEM((d_model, n_experts), jnp.float32),
            pltpu.VMEM((n_tok, 1), jnp.int32),
            pltpu.VMEM((EXPERTS_PER_DEV, d_model, d_ff), jnp.float32),
            pltpu.VMEM((d_model, d_ff), jnp.float32),
            pltpu.VMEM((n_tok, d_ff), jnp.float32),
            pltpu.VMEM((N_STEPS, n_tok, d_ff), jnp.bfloat16),
            pltpu.VMEM((N_STEPS, n_tok, d_ff), jnp.bfloat16),
            pltpu.SemaphoreType.DMA((5,)),
            pltpu.SemaphoreType.DMA((N_STEPS,)),
            pltpu.SemaphoreType.DMA((N_STEPS,)),
        ],
        compiler_params=pltpu.CompilerParams(collective_id=0),
    )(x, router_W, route_idx, expert_W, shared_W)


# device time: 20862 ns/iter; 1.1069x vs baseline; 1.1069x over previous
import jax
import jax.numpy as jnp
from jax import lax
from jax.experimental import pallas as pl
from jax.experimental.pallas import tpu as pltpu

N_DEV = 16
N_STEPS = 4
N_HALF = 2
EXPERTS_PER_DEV = 2


def kernel(x, router_W, route_idx, expert_W, shared_W):
    n_tok, d_model = x.shape
    n_experts = router_W.shape[1]
    d_ff = expert_W.shape[2]
    half = n_tok // N_HALF

    def body(x_ref, router_W_ref, route_idx_ref, expert_W_ref, shared_W_ref,
             out_ref, acc_ref, send_ref, comm_ref, send_sems, recv_sems):
        my = lax.axis_index("i")

        barrier_sem = pltpu.get_barrier_semaphore()
        for k in range(N_STEPS):
            partner = my ^ (1 << k)
            pl.semaphore_signal(
                barrier_sem, inc=1,
                device_id=(partner,), device_id_type=pl.DeviceIdType.MESH,
            )

        xv = x_ref[...]
        xb = xv.astype(jnp.bfloat16)
        scores = jnp.dot(xb, router_W_ref[...].astype(jnp.bfloat16),
                         preferred_element_type=jnp.float32)
        s_max = jnp.max(scores, axis=-1, keepdims=True)
        p = jnp.exp(scores - s_max)
        probs = p / jnp.sum(p, axis=-1, keepdims=True)

        idx = route_idx_ref[...]
        eids = lax.broadcasted_iota(jnp.int32, (n_tok, n_experts), 1)
        p_tok = jnp.sum(jnp.where(idx == eids, probs, 0.0),
                        axis=-1, keepdims=True)

        w0 = expert_W_ref[0].astype(jnp.bfloat16)
        w1 = expert_W_ref[1].astype(jnp.bfloat16)
        coef0 = jnp.where(idx == my * EXPERTS_PER_DEV, p_tok, 0.0)
        coef1 = jnp.where(idx == my * EXPERTS_PER_DEV + 1, p_tok, 0.0)

        def partial_rows(lo):
            y0 = jnp.dot(xb[lo:lo + half], w0,
                         preferred_element_type=jnp.float32)
            y1 = jnp.dot(xb[lo:lo + half], w1,
                         preferred_element_type=jnp.float32)
            return (coef0[lo:lo + half] * y0
                    + coef1[lo:lo + half] * y1)

        def make_rdma(k, h):
            partner = my ^ (1 << k)
            return pltpu.make_async_remote_copy(
                src_ref=send_ref.at[k, h],
                dst_ref=comm_ref.at[k, h],
                send_sem=send_sems.at[k, h],
                recv_sem=recv_sems.at[k, h],
                device_id=(partner,),
                device_id_type=pl.DeviceIdType.MESH,
            )

        part0 = partial_rows(0)
        acc_ref[pl.ds(0, half), :] = part0
        send_ref[0, 0] = part0.astype(jnp.bfloat16)

        pl.semaphore_wait(barrier_sem, N_STEPS)

        rdmas = [[None] * N_HALF for _ in range(N_STEPS)]
        rdmas[0][0] = make_rdma(0, 0)
        rdmas[0][0].start()

        part1 = partial_rows(half)
        acc_ref[pl.ds(half, half), :] = part1
        send_ref[0, 1] = part1.astype(jnp.bfloat16)
        rdmas[0][1] = make_rdma(0, 1)
        rdmas[0][1].start()

        shared = jnp.dot(xb, shared_W_ref[...].astype(jnp.bfloat16),
                         preferred_element_type=jnp.float32)
        out_ref[...] = shared

        for k in range(N_STEPS):
            for h in range(N_HALF):
                sl = pl.ds(h * half, half)
                rdmas[k][h].wait_recv()
                new_acc = acc_ref[sl, :] + comm_ref[k, h].astype(jnp.float32)
                acc_ref[sl, :] = new_acc
                if k + 1 < N_STEPS:
                    send_ref[k + 1, h] = new_acc.astype(jnp.bfloat16)
                    rdmas[k + 1][h] = make_rdma(k + 1, h)
                    rdmas[k + 1][h].start()

        out_ref[...] = out_ref[...] + acc_ref[...]
        for k in range(N_STEPS):
            for h in range(N_HALF):
                rdmas[k][h].wait_send()

    return pl.pallas_call(
        body,
        out_shape=jax.ShapeDtypeStruct((n_tok, d_ff), jnp.float32),
        in_specs=[pl.BlockSpec(memory_space=pltpu.VMEM)] * 5,
        out_specs=pl.BlockSpec(memory_space=pltpu.VMEM),
        scratch_shapes=[
            pltpu.VMEM((n_tok, d_ff), jnp.float32),
            pltpu.VMEM((N_STEPS, N_HALF, half, d_ff), jnp.bfloat16),
            pltpu.VMEM((N_STEPS, N_HALF, half, d_ff), jnp.bfloat16),
            pltpu.SemaphoreType.DMA((N_STEPS, N_HALF)),
            pltpu.SemaphoreType.DMA((N_STEPS, N_HALF)),
        ],
        compiler_params=pltpu.CompilerParams(collective_id=0),
    )(x, router_W, route_idx, expert_W, shared_W)


# device time: 20055 ns/iter; 1.1514x vs baseline; 1.0402x over previous
import jax
import jax.numpy as jnp
from jax import lax
from jax.experimental import pallas as pl
from jax.experimental.pallas import tpu as pltpu

N_DEV = 16
N_STEPS = 4
N_CHUNK = 4
EXPERTS_PER_DEV = 2


def kernel(x, router_W, route_idx, expert_W, shared_W):
    n_tok, d_model = x.shape
    n_experts = router_W.shape[1]
    d_ff = expert_W.shape[2]
    half = n_tok // N_CHUNK

    def body(x_ref, router_W_ref, route_idx_ref, expert_W_ref, shared_W_ref,
             out_ref, acc_ref, send_ref, comm_ref, send_sems, recv_sems):
        my = lax.axis_index("i")

        barrier_sem = pltpu.get_barrier_semaphore()
        for k in range(N_STEPS):
            partner = my ^ (1 << k)
            pl.semaphore_signal(
                barrier_sem, inc=1,
                device_id=(partner,), device_id_type=pl.DeviceIdType.MESH,
            )

        xv = x_ref[...]
        xb = xv.astype(jnp.bfloat16)
        scores = jnp.dot(xb, router_W_ref[...].astype(jnp.bfloat16),
                         preferred_element_type=jnp.float32)
        s_max = jnp.max(scores, axis=-1, keepdims=True)
        p = jnp.exp(scores - s_max)
        probs = p / jnp.sum(p, axis=-1, keepdims=True)

        idx = route_idx_ref[...]
        eids = lax.broadcasted_iota(jnp.int32, (n_tok, n_experts), 1)
        p_tok = jnp.sum(jnp.where(idx == eids, probs, 0.0),
                        axis=-1, keepdims=True)

        w0 = expert_W_ref[0].astype(jnp.bfloat16)
        w1 = expert_W_ref[1].astype(jnp.bfloat16)
        coef0 = jnp.where(idx == my * EXPERTS_PER_DEV, p_tok, 0.0)
        coef1 = jnp.where(idx == my * EXPERTS_PER_DEV + 1, p_tok, 0.0)

        def partial_rows(lo):
            y0 = jnp.dot(xb[lo:lo + half], w0,
                         preferred_element_type=jnp.float32)
            y1 = jnp.dot(xb[lo:lo + half], w1,
                         preferred_element_type=jnp.float32)
            return (coef0[lo:lo + half] * y0
                    + coef1[lo:lo + half] * y1)

        def make_rdma(k, h):
            partner = my ^ (1 << k)
            return pltpu.make_async_remote_copy(
                src_ref=send_ref.at[k, h],
                dst_ref=comm_ref.at[k, h],
                send_sem=send_sems.at[k, h],
                recv_sem=recv_sems.at[k, h],
                device_id=(partner,),
                device_id_type=pl.DeviceIdType.MESH,
            )

        part0 = partial_rows(0)
        acc_ref[pl.ds(0, half), :] = part0
        send_ref[0, 0] = part0.astype(jnp.bfloat16)

        pl.semaphore_wait(barrier_sem, N_STEPS)

        rdmas = [[None] * N_CHUNK for _ in range(N_STEPS)]
        rdmas[0][0] = make_rdma(0, 0)
        rdmas[0][0].start()

        for h in range(1, N_CHUNK):
            part_h = partial_rows(h * half)
            acc_ref[pl.ds(h * half, half), :] = part_h
            send_ref[0, h] = part_h.astype(jnp.bfloat16)
            rdmas[0][h] = make_rdma(0, h)
            rdmas[0][h].start()

        shared = jnp.dot(xb, shared_W_ref[...].astype(jnp.bfloat16),
                         preferred_element_type=jnp.float32)
        out_ref[...] = shared

        for k in range(N_STEPS):
            for h in range(N_CHUNK):
                sl = pl.ds(h * half, half)
                rdmas[k][h].wait_recv()
                new_acc = acc_ref[sl, :] + comm_ref[k, h].astype(jnp.float32)
                acc_ref[sl, :] = new_acc
                if k + 1 < N_STEPS:
                    send_ref[k + 1, h] = new_acc.astype(jnp.bfloat16)
                    rdmas[k + 1][h] = make_rdma(k + 1, h)
                    rdmas[k + 1][h].start()

        out_ref[...] = out_ref[...] + acc_ref[...]
        for k in range(N_STEPS):
            for h in range(N_CHUNK):
                rdmas[k][h].wait_send()

    return pl.pallas_call(
        body,
        out_shape=jax.ShapeDtypeStruct((n_tok, d_ff), jnp.float32),
        in_specs=[pl.BlockSpec(memory_space=pltpu.VMEM)] * 5,
        out_specs=pl.BlockSpec(memory_space=pltpu.VMEM),
        scratch_shapes=[
            pltpu.VMEM((n_tok, d_ff), jnp.float32),
            pltpu.VMEM((N_STEPS, N_CHUNK, half, d_ff), jnp.bfloat16),
            pltpu.VMEM((N_STEPS, N_CHUNK, half, d_ff), jnp.bfloat16),
            pltpu.SemaphoreType.DMA((N_STEPS, N_CHUNK)),
            pltpu.SemaphoreType.DMA((N_STEPS, N_CHUNK)),
        ],
        compiler_params=pltpu.CompilerParams(collective_id=0),
    )(x, router_W, route_idx, expert_W, shared_W)


# device time: 19772 ns/iter; 1.1679x vs baseline; 1.0143x over previous
import jax
import jax.numpy as jnp
from jax import lax
from jax.experimental import pallas as pl
from jax.experimental.pallas import tpu as pltpu

N_DEV = 16
N_STEPS = 4
N_CHUNK = 8
EXPERTS_PER_DEV = 2


def kernel(x, router_W, route_idx, expert_W, shared_W):
    n_tok, d_model = x.shape
    n_experts = router_W.shape[1]
    d_ff = expert_W.shape[2]
    half = n_tok // N_CHUNK

    def body(x_ref, router_W_ref, route_idx_ref, expert_W_ref, shared_W_ref,
             out_ref, acc_ref, send_ref, comm_ref, send_sems, recv_sems):
        my = lax.axis_index("i")

        barrier_sem = pltpu.get_barrier_semaphore()
        for k in range(N_STEPS):
            partner = my ^ (1 << k)
            pl.semaphore_signal(
                barrier_sem, inc=1,
                device_id=(partner,), device_id_type=pl.DeviceIdType.MESH,
            )

        xv = x_ref[...]
        xb = xv.astype(jnp.bfloat16)
        scores = jnp.dot(xb, router_W_ref[...].astype(jnp.bfloat16),
                         preferred_element_type=jnp.float32)
        s_max = jnp.max(scores, axis=-1, keepdims=True)
        p = jnp.exp(scores - s_max)
        probs = p / jnp.sum(p, axis=-1, keepdims=True)

        idx = route_idx_ref[...]
        eids = lax.broadcasted_iota(jnp.int32, (n_tok, n_experts), 1)
        p_tok = jnp.sum(jnp.where(idx == eids, probs, 0.0),
                        axis=-1, keepdims=True)

        w0 = expert_W_ref[0].astype(jnp.bfloat16)
        w1 = expert_W_ref[1].astype(jnp.bfloat16)
        coef0 = jnp.where(idx == my * EXPERTS_PER_DEV, p_tok, 0.0)
        coef1 = jnp.where(idx == my * EXPERTS_PER_DEV + 1, p_tok, 0.0)

        def partial_rows(lo):
            y0 = jnp.dot(xb[lo:lo + half], w0,
                         preferred_element_type=jnp.float32)
            y1 = jnp.dot(xb[lo:lo + half], w1,
                         preferred_element_type=jnp.float32)
            return (coef0[lo:lo + half] * y0
                    + coef1[lo:lo + half] * y1)

        def make_rdma(k, h):
            partner = my ^ (1 << k)
            return pltpu.make_async_remote_copy(
                src_ref=send_ref.at[k, h],
                dst_ref=comm_ref.at[k, h],
                send_sem=send_sems.at[k, h],
                recv_sem=recv_sems.at[k, h],
                device_id=(partner,),
                device_id_type=pl.DeviceIdType.MESH,
            )

        part0 = partial_rows(0)
        acc_ref[pl.ds(0, half), :] = part0
        send_ref[0, 0] = part0.astype(jnp.bfloat16)

        pl.semaphore_wait(barrier_sem, N_STEPS)

        rdmas = [[None] * N_CHUNK for _ in range(N_STEPS)]
        rdmas[0][0] = make_rdma(0, 0)
        rdmas[0][0].start()

        for h in range(1, N_CHUNK):
            part_h = partial_rows(h * half)
            acc_ref[pl.ds(h * half, half), :] = part_h
            send_ref[0, h] = part_h.astype(jnp.bfloat16)
            rdmas[0][h] = make_rdma(0, h)
            rdmas[0][h].start()

        shared = jnp.dot(xb, shared_W_ref[...].astype(jnp.bfloat16),
                         preferred_element_type=jnp.float32)
        out_ref[...] = shared

        for k in range(N_STEPS):
            for h in range(N_CHUNK):
                sl = pl.ds(h * half, half)
                rdmas[k][h].wait_recv()
                new_acc = acc_ref[sl, :] + comm_ref[k, h].astype(jnp.float32)
                acc_ref[sl, :] = new_acc
                if k + 1 < N_STEPS:
                    send_ref[k + 1, h] = new_acc.astype(jnp.bfloat16)
                    rdmas[k + 1][h] = make_rdma(k + 1, h)
                    rdmas[k + 1][h].start()

        out_ref[...] = out_ref[...] + acc_ref[...]
        for k in range(N_STEPS):
            for h in range(N_CHUNK):
                rdmas[k][h].wait_send()

    return pl.pallas_call(
        body,
        out_shape=jax.ShapeDtypeStruct((n_tok, d_ff), jnp.float32),
        in_specs=[pl.BlockSpec(memory_space=pltpu.VMEM)] * 5,
        out_specs=pl.BlockSpec(memory_space=pltpu.VMEM),
        scratch_shapes=[
            pltpu.VMEM((n_tok, d_ff), jnp.float32),
            pltpu.VMEM((N_STEPS, N_CHUNK, half, d_ff), jnp.bfloat16),
            pltpu.VMEM((N_STEPS, N_CHUNK, half, d_ff), jnp.bfloat16),
            pltpu.SemaphoreType.DMA((N_STEPS, N_CHUNK)),
            pltpu.SemaphoreType.DMA((N_STEPS, N_CHUNK)),
        ],
        compiler_params=pltpu.CompilerParams(collective_id=0),
    )(x, router_W, route_idx, expert_W, shared_W)
